# Initial kernel scaffold; baseline (speedup 1.0000x reference)
import jax
import jax.numpy as jnp
from jax.experimental import pallas as pl


def kernel(indices, table):
    raise NotImplementedError("write your pallas kernel here")


# speedup vs baseline: 1.7455x; 1.7455x over previous
import jax
import jax.numpy as jnp
from jax.experimental import pallas as pl
from jax.experimental.pallas import tpu as pltpu
from jax.experimental.pallas import tpu_sc as plsc


def kernel(indices, table):
    B, H = indices.shape
    V, D = table.shape
    N = B * H
    W = 128
    assert N % W == 0

    idx_flat = indices.reshape(1, N).astype(jnp.int32)
    mesh = plsc.VectorSubcoreMesh(core_axis_name="core", subcore_axis_name="subcore")

    @pl.kernel(
        out_type=jax.ShapeDtypeStruct((N, D), table.dtype),
        mesh=mesh,
        compiler_params=pltpu.CompilerParams(use_tc_tiling_on_sc=False),
    )
    def gather_kernel(table_hbm, idx_hbm, out_hbm):
        def body(i_vmem, o_vmem):
            pltpu.sync_copy(table_hbm.at[i_vmem.at[0]], o_vmem)

        pltpu.emit_pipeline(
            body,
            grid=(N // W,),
            in_specs=[pl.BlockSpec((1, W), index_map=lambda i: (0, i))],
            out_specs=[pl.BlockSpec((W, D), index_map=lambda i: (i, 0))],
            core_axis_name=("core", "subcore"),
            dimension_semantics=(pltpu.PARALLEL,),
        )(idx_hbm, out_hbm)

    out = gather_kernel(table, idx_flat)
    return out.reshape(B, H, D)


# speedup vs baseline: 1.8728x; 1.0729x over previous
import jax
import jax.numpy as jnp
from jax import lax
from jax.experimental import pallas as pl
from jax.experimental.pallas import tpu as pltpu
from jax.experimental.pallas import tpu_sc as plsc

NC, NS = 2, 16
NW = NC * NS
W = 128
K = 8


def kernel(indices, table):
    B, H = indices.shape
    V, D = table.shape
    N = B * H
    assert N % (NW * W) == 0
    chunks = N // (NW * W)
    assert chunks % K == 0 and chunks > K

    idx2d = indices.reshape(N // W, W).astype(jnp.int32)
    mesh = plsc.VectorSubcoreMesh(core_axis_name="c", subcore_axis_name="s")

    @pl.kernel(
        out_type=jax.ShapeDtypeStruct((N, D), table.dtype),
        mesh=mesh,
        scratch_types=[
            pltpu.VMEM((chunks, W), jnp.int32),
            pltpu.VMEM((K, W, D), jnp.float32),
            pltpu.SemaphoreType.DMA((K,)),
            pltpu.SemaphoreType.DMA((K,)),
            pltpu.SemaphoreType.DMA,
        ],
        compiler_params=pltpu.CompilerParams(use_tc_tiling_on_sc=False),
    )
    def gather_kernel(table_hbm, idx_hbm, out_hbm, idx_v, rows_v, gsem, osem, isem):
        wid = lax.axis_index("s") * NC + lax.axis_index("c")
        cbase = wid * chunks

        pltpu.async_copy(idx_hbm.at[pl.ds(cbase, chunks)], idx_v, isem).wait()

        def gather_start(b, j):
            pltpu.make_async_copy(
                table_hbm.at[idx_v.at[j]], rows_v.at[b], gsem.at[b]
            ).start()

        def gather_wait(b, j):
            pltpu.make_async_copy(
                table_hbm.at[idx_v.at[j]], rows_v.at[b], gsem.at[b]
            ).wait()

        def out_start(b, j):
            pltpu.make_async_copy(
                rows_v.at[b], out_hbm.at[pl.ds((cbase + j) * W, W)], osem.at[b]
            ).start()

        def out_wait(b, j):
            pltpu.make_async_copy(
                rows_v.at[b], out_hbm.at[pl.ds((cbase + j) * W, W)], osem.at[b]
            ).wait()

        for b in range(K):
            gather_start(b, b)

        @pl.loop(0, chunks - K, step=K)
        def _(j0):
            for b in range(K):
                gather_wait(b, j0 + b)
                out_start(b, j0 + b)
            for b in range(K):
                out_wait(b, j0 + b)
                gather_start(b, j0 + K + b)

        for b in range(K):
            j = chunks - K + b
            gather_wait(b, j)
            out_start(b, j)
            out_wait(b, j)

    out = gather_kernel(table, idx2d)
    return out.reshape(B, H, D)
